# baseline (speedup 1.0000x reference)
import functools

import jax
import jax.numpy as jnp
from jax import lax
from jax.experimental import pallas as pl
from jax.experimental.pallas import tpu as pltpu
from jax.experimental.pallas import tpu_sc as plsc

NBUF = 4
LANES = 16


@functools.lru_cache(maxsize=None)
def _make_sc_kernel(B, S, V, D):
    info = plsc.get_sparse_core_info()
    NC, NS = info.num_cores, info.num_subcores
    NW = NC * NS
    TOTAL = B * S
    PW = TOTAL // NW
    CH = S // 2
    NCH = PW // CH
    KD = D // LANES

    assert TOTAL % NW == 0 and PW % S == 0 and S % 2 == 0
    assert CH <= 128 and NCH % NBUF == 0 and D % LANES == 0

    mesh = plsc.VectorSubcoreMesh(core_axis_name="c", subcore_axis_name="s")

    @functools.partial(
        pl.kernel,
        mesh=mesh,
        compiler_params=pltpu.CompilerParams(use_tc_tiling_on_sc=False),
        out_type=jax.ShapeDtypeStruct((TOTAL * D,), jnp.float32),
        scratch_types=[
            pltpu.VMEM((NCH, CH), jnp.int32),
            pltpu.VMEM((2, CH * D), jnp.float32),
        ]
        + [pltpu.VMEM((CH, D), jnp.float32) for _ in range(NBUF)]
        + [pltpu.VMEM((CH * D,), jnp.float32) for _ in range(NBUF)]
        + [pltpu.SemaphoreType.DMA for _ in range(2 * NBUF)],
    )
    def k(tok_hbm, idx_hbm, pos_hbm, out_hbm, idx_v, pos_v, *bufs_and_sems):
        gbufs = bufs_and_sems[0:NBUF]
        sbufs = bufs_and_sems[NBUF:2 * NBUF]
        gsems = bufs_and_sems[2 * NBUF:3 * NBUF]
        ssems = bufs_and_sems[3 * NBUF:4 * NBUF]

        wid = lax.axis_index("s") * NC + lax.axis_index("c")
        base = wid * PW

        pltpu.sync_copy(idx_hbm.at[wid], idx_v)
        pltpu.sync_copy(pos_hbm, pos_v)

        def gather(c, b):
            return pltpu.make_async_copy(
                tok_hbm.at[idx_v.at[c]], gbufs[b], gsems[b])

        def scatter(c, b):
            return pltpu.make_async_copy(
                sbufs[b],
                out_hbm.at[pl.ds((base + c * CH) * D, CH * D)],
                ssems[b])

        for b in range(NBUF):
            gather(b, b).start()

        def outer(i, carry):
            c0 = i * NBUF
            for b in range(NBUF):
                c = c0 + b
                gather(c, b).wait()

                @pl.when(i > 0)
                def _():
                    scatter(c - NBUF, b).wait()

                par = b % 2

                def addrow(r, carry2, _b=b, _par=par):
                    for kk in range(KD):
                        sl = pl.ds(kk * LANES, LANES)
                        fl = pl.ds(r * D + kk * LANES, LANES)
                        sbufs[_b][fl] = gbufs[_b][r, sl] + pos_v[_par, fl]
                    return carry2

                lax.fori_loop(0, CH, addrow, 0, unroll=4)

                scatter(c, b).start()

                @pl.when(c + NBUF < NCH)
                def _():
                    gather(c + NBUF, b).start()
            return carry

        lax.fori_loop(0, NCH // NBUF, outer, 0)

        for b in range(NBUF):
            scatter(NCH - NBUF + b, b).wait()

    return k, NW, NCH, CH


def kernel(seq, tok_table, pos_table):
    B, S = seq.shape
    V, D = tok_table.shape
    k, NW, NCH, CH = _make_sc_kernel(B, S, V, D)
    idx = seq.astype(jnp.int32).reshape(NW, NCH, CH)
    pos = pos_table.reshape(2, (S // 2) * D)
    out = k(tok_table, idx, pos)
    return out.reshape(B, S, D)


# speedup vs baseline: 1.3087x; 1.3087x over previous
import functools

import jax
import jax.numpy as jnp
from jax import lax
from jax.experimental import pallas as pl
from jax.experimental.pallas import tpu as pltpu
from jax.experimental.pallas import tpu_sc as plsc

NBUF = 8
LOOK = 4
LANES = 16


@functools.lru_cache(maxsize=None)
def _make_sc_kernel(B, S, V, D):
    info = plsc.get_sparse_core_info()
    NC, NS = info.num_cores, info.num_subcores
    NW = NC * NS
    TOTAL = B * S
    PW = TOTAL // NW
    CH = S // 2
    NCH = PW // CH
    KD = D // LANES

    assert TOTAL % NW == 0 and PW % S == 0 and S % 2 == 0
    assert CH <= 128 and NCH % NBUF == 0 and D % LANES == 0

    mesh = plsc.VectorSubcoreMesh(core_axis_name="c", subcore_axis_name="s")

    @functools.partial(
        pl.kernel,
        mesh=mesh,
        compiler_params=pltpu.CompilerParams(use_tc_tiling_on_sc=False),
        out_type=jax.ShapeDtypeStruct((TOTAL, D), jnp.float32),
        scratch_types=[
            pltpu.VMEM((NCH, CH), jnp.int32),
            pltpu.VMEM((2, CH, D), jnp.float32),
        ]
        + [pltpu.VMEM((CH, D), jnp.float32) for _ in range(NBUF)]
        + [pltpu.SemaphoreType.DMA for _ in range(2 * NBUF)],
    )
    def k(tok_hbm, idx_hbm, pos_hbm, out_hbm, idx_v, pos_v, *bufs_and_sems):
        bufs = bufs_and_sems[0:NBUF]
        gsems = bufs_and_sems[NBUF:2 * NBUF]
        ssems = bufs_and_sems[2 * NBUF:3 * NBUF]

        wid = lax.axis_index("s") * NC + lax.axis_index("c")
        base = wid * PW

        pltpu.sync_copy(idx_hbm.at[wid], idx_v)
        pltpu.sync_copy(pos_hbm, pos_v)

        def gather(c, b):
            return pltpu.make_async_copy(
                tok_hbm.at[idx_v.at[c]], bufs[b], gsems[b])

        def scatter(c, b):
            return pltpu.make_async_copy(
                bufs[b],
                out_hbm.at[pl.ds(base + c * CH, CH)],
                ssems[b])

        for b in range(LOOK):
            gather(b, b).start()

        def outer(i, carry):
            c0 = i * NBUF
            for b in range(NBUF):
                c = c0 + b
                fb = (b + LOOK) % NBUF

                @pl.when(c + LOOK < NCH)
                def _(c=c, fb=fb, b=b, i=i):
                    if b >= LOOK:
                        scatter(c - LOOK, fb).wait()
                        gather(c + LOOK, fb).start()
                    else:
                        @pl.when(i > 0)
                        def _():
                            scatter(c - LOOK, fb).wait()
                        gather(c + LOOK, fb).start()

                gather(c, b).wait()

                par = b % 2

                @plsc.parallel_loop(0, CH, unroll=4)
                def addrow(r, _b=b, _par=par):
                    for kk in range(KD):
                        sl = pl.ds(kk * LANES, LANES)
                        plsc.addupdate(bufs[_b].at[r, sl], pos_v[_par, r, sl])

                scatter(c, b).start()
            return carry

        lax.fori_loop(0, NCH // NBUF, outer, 0)

        for b in range(NBUF):
            scatter(NCH - NBUF + b, b).wait()

    return k, NW, NCH, CH


def kernel(seq, tok_table, pos_table):
    B, S = seq.shape
    V, D = tok_table.shape
    k, NW, NCH, CH = _make_sc_kernel(B, S, V, D)
    idx = seq.astype(jnp.int32).reshape(NW, NCH, CH)
    pos = pos_table.reshape(2, S // 2, D)
    out = k(tok_table, idx, pos)
    return out.reshape(B, S, D)
